# Initial kernel scaffold; baseline (speedup 1.0000x reference)
import jax
import jax.numpy as jnp
from jax.experimental import pallas as pl


def kernel(x, edge_index, batch, bn_feat_g, bn_feat_b, conv_feat_W, gin_W1, gin_b1, gin_bng, gin_bnb, gin_W2, gin_b2, fc_bng, fc_bnb, fc_W, fc_b, bnh_g, bnh_b, cls_W, cls_b):
    raise NotImplementedError("write your pallas kernel here")


# speedup vs baseline: 2.6137x; 2.6137x over previous
import functools

import jax
import jax.numpy as jnp
from jax import lax
from jax.experimental import pallas as pl
from jax.experimental.pallas import tpu as pltpu
from jax.experimental.pallas import tpu_sc as plsc

_N = 10000
_D = 128
_G = 64
_EPS = 1e-5

_NC = 2
_NS = 16
_EB = 128
_TB = 80
_EP = _NC * _NS * _TB * _EB
_ACC_ROWS = 10240
_ZROWS = _ACC_ROWS // _NS
_OROWS = 624
_DUMP = _N


def _sc_agg_body(h_hbm, src_hbm, dst_hbm, z_hbm, out_hbm,
                 srcv, dstv, rows, acc, sem):
    c = lax.axis_index("c")
    s = lax.axis_index("s")
    g = c * _NS + s
    pltpu.sync_copy(z_hbm, acc.at[pl.ds(s * _ZROWS, _ZROWS)])
    pltpu.sync_copy(src_hbm.at[pl.ds(g * _TB, _TB)], srcv)
    pltpu.sync_copy(dst_hbm.at[pl.ds(g * _TB, _TB)], dstv)
    plsc.subcore_barrier()

    def step(j, carry):
        pltpu.async_copy(h_hbm.at[srcv.at[j]], rows, sem).wait()
        pltpu.sync_copy(rows, acc.at[dstv.at[j]], add=True)
        return carry

    lax.fori_loop(0, _TB, step, 0)
    plsc.subcore_barrier()

    @pl.when(s < _NS - 1)
    def _():
        pltpu.sync_copy(acc.at[pl.ds(s * _OROWS, _OROWS)],
                        out_hbm.at[c].at[pl.ds(s * _OROWS, _OROWS)])

    @pl.when(s == _NS - 1)
    def _():
        tail = _N - (_NS - 1) * _OROWS
        pltpu.sync_copy(acc.at[pl.ds((_NS - 1) * _OROWS, tail)],
                        out_hbm.at[c].at[pl.ds((_NS - 1) * _OROWS, tail)])


def _sc_segment_sum(h, src2, dst2, zeros):
    mesh = plsc.VectorSubcoreMesh(core_axis_name="c", subcore_axis_name="s")
    f = pl.kernel(
        _sc_agg_body,
        out_type=jax.ShapeDtypeStruct((_NC, _N, _D), jnp.float32),
        mesh=mesh,
        scratch_types=[
            pltpu.VMEM((_TB, _EB), jnp.int32),
            pltpu.VMEM((_TB, _EB), jnp.int32),
            pltpu.VMEM((_EB, _D), jnp.float32),
            pltpu.VMEM_SHARED((_ACC_ROWS, _D), jnp.float32),
            pltpu.SemaphoreType.DMA,
        ],
    )
    return f(h, src2, dst2, zeros)


def _bn_cols(t, g, b):
    mu = jnp.mean(t, axis=0, keepdims=True)
    var = jnp.mean((t - mu) ** 2, axis=0, keepdims=True)
    return (t - mu) * lax.rsqrt(var + _EPS) * g + b


def _feat_body(x_ref, g_ref, b_ref, w_ref, o_ref):
    h = _bn_cols(x_ref[...], g_ref[...], b_ref[...])
    o_ref[...] = jnp.maximum(
        jnp.dot(h, w_ref[...], preferred_element_type=jnp.float32), 0.0)


def _gin_body(h_ref, parts_ref, w1_ref, b1_ref, g1_ref, bb1_ref,
              w2_ref, b2_ref, o_ref):
    parts = parts_ref[...]
    u = h_ref[...] + parts[0] + parts[1]
    t = jnp.dot(u, w1_ref[...], preferred_element_type=jnp.float32) + b1_ref[...]
    t = jnp.maximum(_bn_cols(t, g1_ref[...], bb1_ref[...]), 0.0)
    o_ref[...] = jnp.maximum(
        jnp.dot(t, w2_ref[...], preferred_element_type=jnp.float32)
        + b2_ref[...], 0.0)


def _head_body(h_ref, bat_ref, fg_ref, fb_ref, fw_ref, fb2_ref,
               hg_ref, hb_ref, cw_ref, cb_ref, o_ref):
    gids = lax.broadcasted_iota(jnp.int32, (_G, 1), 0)
    onehot = (bat_ref[...] == gids).astype(jnp.float32)
    p = jnp.dot(onehot, h_ref[...], preferred_element_type=jnp.float32)
    p = _bn_cols(p, fg_ref[...], fb_ref[...])
    p = jnp.maximum(
        jnp.dot(p, fw_ref[...], preferred_element_type=jnp.float32)
        + fb2_ref[...], 0.0)
    p = _bn_cols(p, hg_ref[...], hb_ref[...])
    logits = jnp.dot(p, cw_ref[...], preferred_element_type=jnp.float32) \
        + cb_ref[...]
    m = jnp.max(logits, axis=-1, keepdims=True)
    lse = m + jnp.log(jnp.sum(jnp.exp(logits - m), axis=-1, keepdims=True))
    o_ref[...] = logits - lse


def _tc(body, out_shape, *args):
    return pl.pallas_call(
        body, out_shape=jax.ShapeDtypeStruct(out_shape, jnp.float32))(*args)


def kernel(x, edge_index, batch, bn_feat_g, bn_feat_b, conv_feat_W,
           gin_W1, gin_b1, gin_bng, gin_bnb, gin_W2, gin_b2,
           fc_bng, fc_bnb, fc_W, fc_b, bnh_g, bnh_b, cls_W, cls_b):
    f32 = jnp.float32
    src = edge_index[0].astype(jnp.int32)
    dst = edge_index[1].astype(jnp.int32)
    pad = _EP - src.shape[0]
    src2 = jnp.concatenate([src, jnp.zeros((pad,), jnp.int32)]).reshape(-1, _EB)
    dst2 = jnp.concatenate(
        [dst, jnp.full((pad,), _DUMP, jnp.int32)]).reshape(-1, _EB)
    zeros = jnp.zeros((_ZROWS, _D), f32)
    bat2 = batch.astype(jnp.int32).reshape(1, _N)
    row = lambda v: v.reshape(1, -1).astype(f32)

    h = _tc(_feat_body, (_N, _D),
            x.astype(f32), row(bn_feat_g), row(bn_feat_b), conv_feat_W)
    for i in range(3):
        parts = _sc_segment_sum(h, src2, dst2, zeros)
        h = _tc(_gin_body, (_N, _D), h, parts,
                gin_W1[i], row(gin_b1[i]), row(gin_bng[i]), row(gin_bnb[i]),
                gin_W2[i], row(gin_b2[i]))
    return _tc(_head_body, (_G, 10), h, bat2,
               row(fc_bng), row(fc_bnb), fc_W, row(fc_b),
               row(bnh_g), row(bnh_b), cls_W, row(cls_b))


# speedup vs baseline: 2.9046x; 1.1113x over previous
import functools

import jax
import jax.numpy as jnp
from jax import lax
from jax.experimental import pallas as pl
from jax.experimental.pallas import tpu as pltpu
from jax.experimental.pallas import tpu_sc as plsc

_N = 10000
_D = 128
_G = 64
_EPS = 1e-5

_NC = 2
_NS = 16
_EB = 128
_TB = 80
_EP = _NC * _NS * _TB * _EB
_IDC = 40
_ACC_ROWS = 10112
_ZROWS = _ACC_ROWS // _NS
_OROWS = 624
_DUMP = _N


def _sc_agg_body(h_hbm, src_hbm, dst_hbm, z_hbm, out_hbm,
                 srcv, dstv, rows, acc, sem0, sem1):
    c = lax.axis_index("c")
    s = lax.axis_index("s")
    g = c * _NS + s
    pltpu.sync_copy(z_hbm, acc.at[pl.ds(s * _ZROWS, _ZROWS)])
    plsc.subcore_barrier()

    def gather(j, b, sem):
        return pltpu.async_copy(h_hbm.at[srcv.at[j]], rows.at[b], sem)

    def scatter(j, b):
        pltpu.sync_copy(rows.at[b], acc.at[dstv.at[j]], add=True)

    n2 = _IDC // 2
    for hh in range(_TB // _IDC):
        pltpu.sync_copy(src_hbm.at[pl.ds(g * _TB + hh * _IDC, _IDC)], srcv)
        pltpu.sync_copy(dst_hbm.at[pl.ds(g * _TB + hh * _IDC, _IDC)], dstv)
        gather(0, 0, sem0)

        def step(i, carry):
            j = i * 2
            gather(j + 1, 1, sem1)
            pltpu.make_async_copy(
                h_hbm.at[srcv.at[j]], rows.at[0], sem0).wait()
            scatter(j, 0)

            @pl.when(i < n2 - 1)
            def _():
                gather(j + 2, 0, sem0)

            pltpu.make_async_copy(
                h_hbm.at[srcv.at[j]], rows.at[1], sem1).wait()
            scatter(j + 1, 1)
            return carry

        lax.fori_loop(0, n2, step, 0)
    plsc.subcore_barrier()

    @pl.when(s < _NS - 1)
    def _():
        pltpu.sync_copy(acc.at[pl.ds(s * _OROWS, _OROWS)],
                        out_hbm.at[c].at[pl.ds(s * _OROWS, _OROWS)])

    @pl.when(s == _NS - 1)
    def _():
        tail = _N - (_NS - 1) * _OROWS
        pltpu.sync_copy(acc.at[pl.ds((_NS - 1) * _OROWS, tail)],
                        out_hbm.at[c].at[pl.ds((_NS - 1) * _OROWS, tail)])


def _sc_segment_sum(h, src2, dst2, zeros):
    mesh = plsc.VectorSubcoreMesh(core_axis_name="c", subcore_axis_name="s")
    f = pl.kernel(
        _sc_agg_body,
        out_type=jax.ShapeDtypeStruct((_NC, _N, _D), jnp.float32),
        mesh=mesh,
        scratch_types=[
            pltpu.VMEM((_IDC, _EB), jnp.int32),
            pltpu.VMEM((_IDC, _EB), jnp.int32),
            pltpu.VMEM((2, _EB, _D), jnp.float32),
            pltpu.VMEM_SHARED((_ACC_ROWS, _D), jnp.float32),
            pltpu.SemaphoreType.DMA,
            pltpu.SemaphoreType.DMA,
        ],
    )
    return f(h, src2, dst2, zeros)


def _bn_cols(t, g, b):
    mu = jnp.mean(t, axis=0, keepdims=True)
    var = jnp.mean((t - mu) ** 2, axis=0, keepdims=True)
    return (t - mu) * lax.rsqrt(var + _EPS) * g + b


def _feat_body(x_ref, g_ref, b_ref, w_ref, o_ref):
    h = _bn_cols(x_ref[...], g_ref[...], b_ref[...])
    o_ref[...] = jnp.maximum(
        jnp.dot(h, w_ref[...], preferred_element_type=jnp.float32), 0.0)


def _gin_body(h_ref, parts_ref, w1_ref, b1_ref, g1_ref, bb1_ref,
              w2_ref, b2_ref, o_ref):
    parts = parts_ref[...]
    u = h_ref[...] + parts[0] + parts[1]
    t = jnp.dot(u, w1_ref[...], preferred_element_type=jnp.float32) + b1_ref[...]
    t = jnp.maximum(_bn_cols(t, g1_ref[...], bb1_ref[...]), 0.0)
    o_ref[...] = jnp.maximum(
        jnp.dot(t, w2_ref[...], preferred_element_type=jnp.float32)
        + b2_ref[...], 0.0)


def _head_body(h_ref, bat_ref, fg_ref, fb_ref, fw_ref, fb2_ref,
               hg_ref, hb_ref, cw_ref, cb_ref, o_ref):
    gids = lax.broadcasted_iota(jnp.int32, (_G, 1), 0)
    onehot = (bat_ref[...] == gids).astype(jnp.float32)
    p = jnp.dot(onehot, h_ref[...], preferred_element_type=jnp.float32)
    p = _bn_cols(p, fg_ref[...], fb_ref[...])
    p = jnp.maximum(
        jnp.dot(p, fw_ref[...], preferred_element_type=jnp.float32)
        + fb2_ref[...], 0.0)
    p = _bn_cols(p, hg_ref[...], hb_ref[...])
    logits = jnp.dot(p, cw_ref[...], preferred_element_type=jnp.float32) \
        + cb_ref[...]
    m = jnp.max(logits, axis=-1, keepdims=True)
    lse = m + jnp.log(jnp.sum(jnp.exp(logits - m), axis=-1, keepdims=True))
    o_ref[...] = logits - lse


def _tc(body, out_shape, *args):
    return pl.pallas_call(
        body, out_shape=jax.ShapeDtypeStruct(out_shape, jnp.float32))(*args)


def kernel(x, edge_index, batch, bn_feat_g, bn_feat_b, conv_feat_W,
           gin_W1, gin_b1, gin_bng, gin_bnb, gin_W2, gin_b2,
           fc_bng, fc_bnb, fc_W, fc_b, bnh_g, bnh_b, cls_W, cls_b):
    f32 = jnp.float32
    src = edge_index[0].astype(jnp.int32)
    dst = edge_index[1].astype(jnp.int32)
    pad = _EP - src.shape[0]
    src2 = jnp.concatenate([src, jnp.zeros((pad,), jnp.int32)]).reshape(-1, _EB)
    dst2 = jnp.concatenate(
        [dst, jnp.full((pad,), _DUMP, jnp.int32)]).reshape(-1, _EB)
    zeros = jnp.zeros((_ZROWS, _D), f32)
    bat2 = batch.astype(jnp.int32).reshape(1, _N)
    row = lambda v: v.reshape(1, -1).astype(f32)

    h = _tc(_feat_body, (_N, _D),
            x.astype(f32), row(bn_feat_g), row(bn_feat_b), conv_feat_W)
    for i in range(3):
        parts = _sc_segment_sum(h, src2, dst2, zeros)
        h = _tc(_gin_body, (_N, _D), h, parts,
                gin_W1[i], row(gin_b1[i]), row(gin_bng[i]), row(gin_bnb[i]),
                gin_W2[i], row(gin_b2[i]))
    return _tc(_head_body, (_G, 10), h, bat2,
               row(fc_bng), row(fc_bnb), fc_W, row(fc_b),
               row(bnh_g), row(bnh_b), cls_W, row(cls_b))


# speedup vs baseline: 2.9075x; 1.0010x over previous
import functools

import jax
import jax.numpy as jnp
from jax import lax
from jax.experimental import pallas as pl
from jax.experimental.pallas import tpu as pltpu
from jax.experimental.pallas import tpu_sc as plsc

_N = 10000
_D = 128
_G = 64
_EPS = 1e-5

_NC = 2
_NS = 16
_EB = 128
_TB = 80
_EP = _NC * _NS * _TB * _EB
_IDC = 40
_ACC_ROWS = 10112
_ZROWS = _ACC_ROWS // _NS
_OROWS = 624
_DUMP = _N


def _sc_agg_body(h_hbm, src_hbm, dst_hbm, z_hbm, out_hbm,
                 srcv, dstv, rows, acc, sem0, sem1):
    c = lax.axis_index("c")
    s = lax.axis_index("s")
    g = c * _NS + s
    pltpu.sync_copy(z_hbm, acc.at[pl.ds(s * _ZROWS, _ZROWS)])
    plsc.subcore_barrier()

    def gather(j, b, sem):
        return pltpu.async_copy(h_hbm.at[srcv.at[j]], rows.at[b], sem)

    def scatter(j, b):
        pltpu.sync_copy(rows.at[b], acc.at[dstv.at[j]], add=True)

    n2 = _IDC // 2
    for hh in range(_TB // _IDC):
        pltpu.sync_copy(src_hbm.at[pl.ds(g * _TB + hh * _IDC, _IDC)], srcv)
        pltpu.sync_copy(dst_hbm.at[pl.ds(g * _TB + hh * _IDC, _IDC)], dstv)
        gather(0, 0, sem0)

        def step(i, carry):
            j = i * 2
            gather(j + 1, 1, sem1)
            pltpu.make_async_copy(
                h_hbm.at[srcv.at[j]], rows.at[0], sem0).wait()
            scatter(j, 0)

            @pl.when(i < n2 - 1)
            def _():
                gather(j + 2, 0, sem0)

            pltpu.make_async_copy(
                h_hbm.at[srcv.at[j]], rows.at[1], sem1).wait()
            scatter(j + 1, 1)
            return carry

        lax.fori_loop(0, n2, step, 0)
    plsc.subcore_barrier()

    @pl.when(s < _NS - 1)
    def _():
        pltpu.sync_copy(acc.at[pl.ds(s * _OROWS, _OROWS)],
                        out_hbm.at[c].at[pl.ds(s * _OROWS, _OROWS)])

    @pl.when(s == _NS - 1)
    def _():
        tail = _N - (_NS - 1) * _OROWS
        pltpu.sync_copy(acc.at[pl.ds((_NS - 1) * _OROWS, tail)],
                        out_hbm.at[c].at[pl.ds((_NS - 1) * _OROWS, tail)])


def _sc_segment_sum(h, src2, dst2, zeros):
    mesh = plsc.VectorSubcoreMesh(core_axis_name="c", subcore_axis_name="s")
    f = pl.kernel(
        _sc_agg_body,
        out_type=jax.ShapeDtypeStruct((_NC, _N, _D), jnp.float32),
        mesh=mesh,
        scratch_types=[
            pltpu.VMEM((_IDC, _EB), jnp.int32),
            pltpu.VMEM((_IDC, _EB), jnp.int32),
            pltpu.VMEM((2, _EB, _D), jnp.float32),
            pltpu.VMEM_SHARED((_ACC_ROWS, _D), jnp.float32),
            pltpu.SemaphoreType.DMA,
            pltpu.SemaphoreType.DMA,
        ],
    )
    return f(h, src2, dst2, zeros)


def _bn_cols(t, g, b):
    mu = jnp.mean(t, axis=0, keepdims=True)
    var = jnp.mean((t - mu) ** 2, axis=0, keepdims=True)
    return (t - mu) * lax.rsqrt(var + _EPS) * g + b


def _feat_body(x_ref, g_ref, b_ref, w_ref, o_ref):
    h = _bn_cols(x_ref[...], g_ref[...], b_ref[...])
    o_ref[...] = jnp.maximum(
        jnp.dot(h, w_ref[...], preferred_element_type=jnp.float32), 0.0)


def _gin_body(h_ref, parts_ref, w1_ref, b1_ref, g1_ref, bb1_ref,
              w2_ref, b2_ref, o_ref):
    parts = parts_ref[...]
    u = h_ref[...] + parts[0] + parts[1]
    t = jnp.dot(u, w1_ref[...], preferred_element_type=jnp.float32) + b1_ref[...]
    t = jnp.maximum(_bn_cols(t, g1_ref[...], bb1_ref[...]), 0.0)
    o_ref[...] = jnp.maximum(
        jnp.dot(t, w2_ref[...], preferred_element_type=jnp.float32)
        + b2_ref[...], 0.0)


def _head_body(h_ref, bat_ref, fg_ref, fb_ref, fw_ref, fb2_ref,
               hg_ref, hb_ref, cw_ref, cb_ref, o_ref):
    gids = lax.broadcasted_iota(jnp.int32, (_G, 1), 0)
    onehot = (bat_ref[...] == gids).astype(jnp.float32)
    p = jnp.dot(onehot, h_ref[...], preferred_element_type=jnp.float32)
    p = _bn_cols(p, fg_ref[...], fb_ref[...])
    p = jnp.maximum(
        jnp.dot(p, fw_ref[...], preferred_element_type=jnp.float32)
        + fb2_ref[...], 0.0)
    p = _bn_cols(p, hg_ref[...], hb_ref[...])
    logits = jnp.dot(p, cw_ref[...], preferred_element_type=jnp.float32) \
        + cb_ref[...]
    m = jnp.max(logits, axis=-1, keepdims=True)
    lse = m + jnp.log(jnp.sum(jnp.exp(logits - m), axis=-1, keepdims=True))
    o_ref[...] = logits - lse


def _tc(body, out_shape, *args):
    return pl.pallas_call(
        body, out_shape=jax.ShapeDtypeStruct(out_shape, jnp.float32))(*args)


def kernel(x, edge_index, batch, bn_feat_g, bn_feat_b, conv_feat_W,
           gin_W1, gin_b1, gin_bng, gin_bnb, gin_W2, gin_b2,
           fc_bng, fc_bnb, fc_W, fc_b, bnh_g, bnh_b, cls_W, cls_b):
    f32 = jnp.float32
    src = edge_index[0].astype(jnp.int32)
    dst = edge_index[1].astype(jnp.int32)
    pad = _EP - src.shape[0]
    dummy = _DUMP + jnp.arange(pad, dtype=jnp.int32) % (_ACC_ROWS - _DUMP)
    src2 = jnp.concatenate([src, jnp.zeros((pad,), jnp.int32)]).reshape(-1, _EB)
    dst2 = jnp.concatenate([dst, dummy]).reshape(-1, _EB)
    zeros = jnp.zeros((_ZROWS, _D), f32)
    bat2 = batch.astype(jnp.int32).reshape(1, _N)
    row = lambda v: v.reshape(1, -1).astype(f32)

    h = _tc(_feat_body, (_N, _D),
            x.astype(f32), row(bn_feat_g), row(bn_feat_b), conv_feat_W)
    for i in range(3):
        parts = _sc_segment_sum(h, src2, dst2, zeros)
        h = _tc(_gin_body, (_N, _D), h, parts,
                gin_W1[i], row(gin_b1[i]), row(gin_bng[i]), row(gin_bnb[i]),
                gin_W2[i], row(gin_b2[i]))
    return _tc(_head_body, (_G, 10), h, bat2,
               row(fc_bng), row(fc_bnb), fc_W, row(fc_b),
               row(bnh_g), row(bnh_b), cls_W, row(cls_b))
